# baseline (speedup 1.0000x reference)
import functools

import jax
import jax.numpy as jnp
from jax import lax
from jax.experimental import pallas as pl
from jax.experimental.pallas import tpu as pltpu
from jax.experimental.pallas import tpu_sc as plsc

D = 1024
EPS = 1e-5
NW = 32
G = 32
L = 16
NL = D // L


def _lane_sum(x):
    dnums = lax.GatherDimensionNumbers(
        offset_dims=(), collapsed_slice_dims=(0,), start_index_map=(0,))
    for sh in (8, 4, 2, 1):
        perm = lax.iota(jnp.int32, L) ^ sh
        x = x + lax.gather(x, perm[:, None], dnums, (1,),
                           mode=lax.GatherScatterMode.PROMISE_IN_BOUNDS)
    return x


def _body(idx_hbm, tok_hbm, pos_hbm, gam_hbm, bet_hbm, out_hbm,
          idx_v, tbuf, pbuf, gam_v, bet_v, sem, *, nch, seq):
    nc = 2
    wid = lax.axis_index("s") * nc + lax.axis_index("c")
    tpw = nch * G
    base = wid * tpw
    s_off = (wid % (seq // tpw)) * tpw

    pltpu.sync_copy(idx_hbm.at[wid], idx_v)
    pltpu.sync_copy(gam_hbm, gam_v)
    pltpu.sync_copy(bet_hbm, bet_v)

    def chunk_body(k, carry):
        pltpu.sync_copy(pos_hbm.at[pl.ds(s_off + k * G, G)], pbuf)
        pltpu.async_copy(tok_hbm.at[idx_v.at[k]], tbuf, sem).wait()

        def row_body(r, rcarry):
            acc_s = jnp.zeros((L,), jnp.float32)
            acc_q = jnp.zeros((L,), jnp.float32)
            for c in range(NL):
                sl = pl.ds(c * L, L)
                v = tbuf[r, sl] + pbuf[r, sl]
                tbuf[r, sl] = v
                acc_s = acc_s + v
                acc_q = acc_q + v * v
            mean = _lane_sum(acc_s) * (1.0 / D)
            var = _lane_sum(acc_q) * (1.0 / D) - mean * mean
            xv = var + EPS
            bits = plsc.bitcast(xv, jnp.int32)
            bits = jnp.int32(0x5F3759DF) - (bits >> 1)
            y = plsc.bitcast(bits, jnp.float32)
            for _ in range(3):
                y = y * (1.5 - 0.5 * xv * y * y)
            for c in range(NL):
                sl = pl.ds(c * L, L)
                v = tbuf[r, sl]
                tbuf[r, sl] = (v - mean) * y * gam_v[sl] + bet_v[sl]
            return rcarry

        lax.fori_loop(0, G, row_body, 0)
        pltpu.sync_copy(tbuf, out_hbm.at[pl.ds(base + k * G, G)])
        return carry

    lax.fori_loop(0, nch, chunk_body, 0)


def kernel(x, token_table, pos_table, gamma, beta):
    b, s = x.shape
    n_tok = b * s
    tpw = n_tok // NW
    nch = tpw // G
    idx = x.reshape(NW, nch, G).astype(jnp.int32)

    mesh = plsc.VectorSubcoreMesh(core_axis_name="c", subcore_axis_name="s")
    run = pl.kernel(
        functools.partial(_body, nch=nch, seq=s),
        out_type=jax.ShapeDtypeStruct((n_tok, D), jnp.float32),
        mesh=mesh,
        compiler_params=pltpu.CompilerParams(needs_layout_passes=False),
        scratch_types=[
            pltpu.VMEM((nch, G), jnp.int32),
            pltpu.VMEM((G, D), jnp.float32),
            pltpu.VMEM((G, D), jnp.float32),
            pltpu.VMEM((D,), jnp.float32),
            pltpu.VMEM((D,), jnp.float32),
            pltpu.SemaphoreType.DMA,
        ],
    )
    out = run(idx, token_table, pos_table, gamma, beta)
    return out.reshape(b, s, D)
